# baseline (speedup 1.0000x reference)
import functools

import jax
import jax.numpy as jnp
from jax import lax
from jax.experimental import pallas as pl
from jax.experimental.pallas import tpu as pltpu
from jax.experimental.pallas import tpu_sc as plsc

N_DAYS = 366
D_MODEL = 256
BATCH = 16384

NC = 2
NS = 16
NW = NC * NS
B_PER_W = BATCH // NW
CHUNK = 128
N_CHUNK = B_PER_W // CHUNK
NBUF = 3
NREP = 4
N_DAYS_PAD = 368
STAGE = 96

_mesh = plsc.VectorSubcoreMesh(core_axis_name="c", subcore_axis_name="s")


@functools.partial(
    pl.kernel,
    mesh=_mesh,
    out_type=jax.ShapeDtypeStruct((BATCH, D_MODEL), jnp.float32),
    scratch_types=[
        pltpu.VMEM((N_CHUNK, CHUNK), jnp.int32),
        pltpu.HBM((NC * NREP * N_DAYS_PAD, D_MODEL), jnp.float32),
        *[pltpu.VMEM((CHUNK, D_MODEL), jnp.float32) for _ in range(NBUF)],
        pltpu.SemaphoreType.DMA,
        *[pltpu.SemaphoreType.DMA for _ in range(2 * NBUF)],
    ],
)
def _gather_kernel(x_hbm, pe_hbm, out_hbm, idx_v, repl_hbm, *scratch):
    bufs = scratch[:NBUF]
    isem = scratch[NBUF]
    gsems = scratch[NBUF + 1:NBUF + 1 + NBUF]
    ssems = scratch[NBUF + 1 + NBUF:]
    sid = lax.axis_index("s")
    cid = lax.axis_index("c")
    wid = sid * NC + cid
    base = wid * B_PER_W

    icp = [
        pltpu.async_copy(x_hbm.at[pl.ds(base + j * CHUNK, CHUNK)], idx_v.at[j], isem)
        for j in range(N_CHUNK)
    ]

    rep = sid // NREP
    strip = sid % NREP
    dst0 = (cid * NREP + rep) * N_DAYS_PAD + strip * STAGE
    LAST = N_DAYS_PAD - (NREP - 1) * STAGE

    @pl.when(strip < NREP - 1)
    def _():
        pltpu.sync_copy(
            pe_hbm.at[pl.ds(strip * STAGE, STAGE)], repl_hbm.at[pl.ds(dst0, STAGE)]
        )

    @pl.when(strip == NREP - 1)
    def _():
        pltpu.sync_copy(
            pe_hbm.at[pl.ds((NREP - 1) * STAGE, LAST)], repl_hbm.at[pl.ds(dst0, LAST)]
        )

    for cp in icp:
        cp.wait()

    roff = (cid * NREP + sid % NREP) * N_DAYS_PAD
    for j in range(N_CHUNK):
        for k in range(CHUNK // 16):
            v = idx_v[j, pl.ds(k * 16, 16)]
            idx_v[j, pl.ds(k * 16, 16)] = jnp.where(v == 0, N_DAYS - 1, v - 1) + roff

    plsc.subcore_barrier()

    def gather(j):
        return pltpu.async_copy(repl_hbm.at[idx_v.at[j]], bufs[j % NBUF], gsems[j % NBUF])

    def scatter(j):
        return pltpu.async_copy(
            bufs[j % NBUF], out_hbm.at[pl.ds(base + j * CHUNK, CHUNK)], ssems[j % NBUF]
        )

    gcp = [None] * N_CHUNK
    scp = [None] * N_CHUNK
    for j in range(min(NBUF, N_CHUNK)):
        gcp[j] = gather(j)
    for j in range(N_CHUNK):
        gcp[j].wait()
        scp[j] = scatter(j)
        if j + NBUF < N_CHUNK:
            scp[j].wait()
            gcp[j + NBUF] = gather(j + NBUF)
    for j in range(max(0, N_CHUNK - NBUF), N_CHUNK):
        scp[j].wait()


def kernel(x, pe):
    pe_pad = jnp.pad(pe, ((0, N_DAYS_PAD - N_DAYS), (0, 0)))
    return _gather_kernel(x.astype(jnp.int32), pe_pad)


# speedup vs baseline: 3.2773x; 3.2773x over previous
import functools

import jax
import jax.numpy as jnp
from jax import lax
from jax.experimental import pallas as pl
from jax.experimental.pallas import tpu as pltpu
from jax.experimental.pallas import tpu_sc as plsc

N_DAYS = 366
D_MODEL = 256
BATCH = 16384

NC = 2
NS = 16
NW = NC * NS
B_PER_W = BATCH // NW
CHUNK = 128
N_CHUNK = B_PER_W // CHUNK
NBUF = 3
NREP = 4
N_DAYS_PAD = 368
STAGE = 96

_mesh = plsc.VectorSubcoreMesh(core_axis_name="c", subcore_axis_name="s")


@functools.partial(
    pl.kernel,
    mesh=_mesh,
    out_type=jax.ShapeDtypeStruct((BATCH, D_MODEL), jnp.float32),
    scratch_types=[
        pltpu.VMEM((N_CHUNK, CHUNK), jnp.int32),
        pltpu.VMEM_SHARED((N_DAYS_PAD, D_MODEL), jnp.float32),
        pltpu.HBM((NC * NREP * N_DAYS_PAD, D_MODEL), jnp.float32),
        *[pltpu.VMEM((CHUNK, D_MODEL), jnp.float32) for _ in range(NBUF)],
        pltpu.SemaphoreType.DMA,
        *[pltpu.SemaphoreType.DMA for _ in range(2 * NBUF)],
    ],
)
def _gather_kernel(x_hbm, pe_hbm, out_hbm, idx_v, table_sh, repl_hbm, *scratch):
    bufs = scratch[:NBUF]
    isem = scratch[NBUF]
    gsems = scratch[NBUF + 1:NBUF + 1 + NBUF]
    ssems = scratch[NBUF + 1 + NBUF:]
    sid = lax.axis_index("s")
    cid = lax.axis_index("c")
    wid = sid * NC + cid
    base = wid * B_PER_W

    icp = [
        pltpu.async_copy(x_hbm.at[pl.ds(base + j * CHUNK, CHUNK)], idx_v.at[j], isem)
        for j in range(N_CHUNK)
    ]

    @pl.when(sid < NS - 1)
    def _():
        pltpu.sync_copy(
            pe_hbm.at[pl.ds(sid * 24, 24)], table_sh.at[pl.ds(sid * 24, 24)]
        )

    @pl.when(sid == NS - 1)
    def _():
        pltpu.sync_copy(
            pe_hbm.at[pl.ds((NS - 1) * 24, N_DAYS_PAD - (NS - 1) * 24)],
            table_sh.at[pl.ds((NS - 1) * 24, N_DAYS_PAD - (NS - 1) * 24)],
        )

    plsc.subcore_barrier()

    rep = sid // NREP
    strip = sid % NREP
    dst0 = (cid * NREP + rep) * N_DAYS_PAD + strip * STAGE
    LAST = N_DAYS_PAD - (NREP - 1) * STAGE

    @pl.when(strip < NREP - 1)
    def _():
        pltpu.sync_copy(
            table_sh.at[pl.ds(strip * STAGE, STAGE)], repl_hbm.at[pl.ds(dst0, STAGE)]
        )

    @pl.when(strip == NREP - 1)
    def _():
        pltpu.sync_copy(
            table_sh.at[pl.ds((NREP - 1) * STAGE, LAST)], repl_hbm.at[pl.ds(dst0, LAST)]
        )

    for cp in icp:
        cp.wait()

    roff = (cid * NREP + sid % NREP) * N_DAYS_PAD
    for j in range(N_CHUNK):
        for k in range(CHUNK // 16):
            v = idx_v[j, pl.ds(k * 16, 16)]
            idx_v[j, pl.ds(k * 16, 16)] = jnp.where(v == 0, N_DAYS - 1, v - 1) + roff

    plsc.subcore_barrier()

    def gather(j):
        return pltpu.async_copy(repl_hbm.at[idx_v.at[j]], bufs[j % NBUF], gsems[j % NBUF])

    def scatter(j):
        return pltpu.async_copy(
            bufs[j % NBUF], out_hbm.at[pl.ds(base + j * CHUNK, CHUNK)], ssems[j % NBUF]
        )

    gcp = [None] * N_CHUNK
    scp = [None] * N_CHUNK
    for j in range(min(NBUF, N_CHUNK)):
        gcp[j] = gather(j)
    for j in range(N_CHUNK):
        gcp[j].wait()
        scp[j] = scatter(j)
        if j + NBUF < N_CHUNK:
            scp[j].wait()
            gcp[j + NBUF] = gather(j + NBUF)
    for j in range(max(0, N_CHUNK - NBUF), N_CHUNK):
        scp[j].wait()


def kernel(x, pe):
    pe_pad = jnp.pad(pe, ((0, N_DAYS_PAD - N_DAYS), (0, 0)))
    return _gather_kernel(x.astype(jnp.int32), pe_pad)


# speedup vs baseline: 3.5322x; 1.0778x over previous
import functools

import jax
import jax.numpy as jnp
from jax import lax
from jax.experimental import pallas as pl
from jax.experimental.pallas import tpu as pltpu
from jax.experimental.pallas import tpu_sc as plsc

N_DAYS = 366
D_MODEL = 256
BATCH = 16384

NC = 2
NS = 16
NW = NC * NS
B_PER_W = BATCH // NW
CHUNK = 128
N_CHUNK = B_PER_W // CHUNK
NBUF = 3
NREP = 8
N_DAYS_PAD = 368

_mesh = plsc.VectorSubcoreMesh(core_axis_name="c", subcore_axis_name="s")


@functools.partial(
    pl.kernel,
    mesh=_mesh,
    out_type=jax.ShapeDtypeStruct((BATCH, D_MODEL), jnp.float32),
    scratch_types=[
        pltpu.VMEM((N_CHUNK, CHUNK), jnp.int32),
        *[pltpu.VMEM((CHUNK, D_MODEL), jnp.float32) for _ in range(NBUF)],
        pltpu.SemaphoreType.DMA,
        *[pltpu.SemaphoreType.DMA for _ in range(2 * NBUF)],
    ],
)
def _gather_kernel(x_hbm, repl_hbm, out_hbm, idx_v, *scratch):
    bufs = scratch[:NBUF]
    isem = scratch[NBUF]
    gsems = scratch[NBUF + 1:NBUF + 1 + NBUF]
    ssems = scratch[NBUF + 1 + NBUF:]
    sid = lax.axis_index("s")
    wid = sid * NC + lax.axis_index("c")
    base = wid * B_PER_W

    icp = [
        pltpu.async_copy(x_hbm.at[pl.ds(base + j * CHUNK, CHUNK)], idx_v.at[j], isem)
        for j in range(N_CHUNK)
    ]
    for cp in icp:
        cp.wait()

    roff = (wid % NREP) * N_DAYS_PAD
    for j in range(N_CHUNK):
        for k in range(CHUNK // 16):
            v = idx_v[j, pl.ds(k * 16, 16)]
            idx_v[j, pl.ds(k * 16, 16)] = jnp.where(v == 0, N_DAYS - 1, v - 1) + roff

    def gather(j):
        return pltpu.async_copy(repl_hbm.at[idx_v.at[j]], bufs[j % NBUF], gsems[j % NBUF])

    def scatter(j):
        return pltpu.async_copy(
            bufs[j % NBUF], out_hbm.at[pl.ds(base + j * CHUNK, CHUNK)], ssems[j % NBUF]
        )

    gcp = [None] * N_CHUNK
    scp = [None] * N_CHUNK
    for j in range(min(NBUF, N_CHUNK)):
        gcp[j] = gather(j)
    for j in range(N_CHUNK):
        gcp[j].wait()
        scp[j] = scatter(j)
        if j + NBUF < N_CHUNK:
            scp[j].wait()
            gcp[j + NBUF] = gather(j + NBUF)
    for j in range(max(0, N_CHUNK - NBUF), N_CHUNK):
        scp[j].wait()


def kernel(x, pe):
    pe_pad = jnp.pad(pe, ((0, N_DAYS_PAD - N_DAYS), (0, 0)))
    repl = jnp.tile(pe_pad, (NREP, 1))
    return _gather_kernel(x.astype(jnp.int32), repl)


# speedup vs baseline: 3.5341x; 1.0005x over previous
import functools

import jax
import jax.numpy as jnp
from jax import lax
from jax.experimental import pallas as pl
from jax.experimental.pallas import tpu as pltpu
from jax.experimental.pallas import tpu_sc as plsc

N_DAYS = 366
D_MODEL = 256
BATCH = 16384

NC = 2
NS = 16
NW = NC * NS
B_PER_W = BATCH // NW
CHUNK = 128
N_CHUNK = B_PER_W // CHUNK
NBUF = 3
NREP = 16
N_DAYS_PAD = 368

_mesh = plsc.VectorSubcoreMesh(core_axis_name="c", subcore_axis_name="s")


@functools.partial(
    pl.kernel,
    mesh=_mesh,
    out_type=jax.ShapeDtypeStruct((BATCH, D_MODEL), jnp.float32),
    scratch_types=[
        pltpu.VMEM((N_CHUNK, CHUNK), jnp.int32),
        *[pltpu.VMEM((CHUNK, D_MODEL), jnp.float32) for _ in range(NBUF)],
        pltpu.SemaphoreType.DMA,
        *[pltpu.SemaphoreType.DMA for _ in range(2 * NBUF)],
    ],
)
def _gather_kernel(x_hbm, repl_hbm, out_hbm, idx_v, *scratch):
    bufs = scratch[:NBUF]
    isem = scratch[NBUF]
    gsems = scratch[NBUF + 1:NBUF + 1 + NBUF]
    ssems = scratch[NBUF + 1 + NBUF:]
    sid = lax.axis_index("s")
    wid = sid * NC + lax.axis_index("c")
    base = wid * B_PER_W

    icp = [
        pltpu.async_copy(x_hbm.at[pl.ds(base + j * CHUNK, CHUNK)], idx_v.at[j], isem)
        for j in range(N_CHUNK)
    ]
    for cp in icp:
        cp.wait()

    roff = (wid % NREP) * N_DAYS_PAD
    for j in range(N_CHUNK):
        for k in range(CHUNK // 16):
            v = idx_v[j, pl.ds(k * 16, 16)]
            idx_v[j, pl.ds(k * 16, 16)] = jnp.where(v == 0, N_DAYS - 1, v - 1) + roff

    def gather(j):
        return pltpu.async_copy(repl_hbm.at[idx_v.at[j]], bufs[j % NBUF], gsems[j % NBUF])

    def scatter(j):
        return pltpu.async_copy(
            bufs[j % NBUF], out_hbm.at[pl.ds(base + j * CHUNK, CHUNK)], ssems[j % NBUF]
        )

    gcp = [None] * N_CHUNK
    scp = [None] * N_CHUNK
    for j in range(min(NBUF, N_CHUNK)):
        gcp[j] = gather(j)
    for j in range(N_CHUNK):
        gcp[j].wait()
        scp[j] = scatter(j)
        if j + NBUF < N_CHUNK:
            scp[j].wait()
            gcp[j + NBUF] = gather(j + NBUF)
    for j in range(max(0, N_CHUNK - NBUF), N_CHUNK):
        scp[j].wait()


def kernel(x, pe):
    pe_pad = jnp.pad(pe, ((0, N_DAYS_PAD - N_DAYS), (0, 0)))
    repl = jnp.tile(pe_pad, (NREP, 1))
    return _gather_kernel(x.astype(jnp.int32), repl)


# speedup vs baseline: 3.5630x; 1.0082x over previous
import functools

import jax
import jax.numpy as jnp
from jax import lax
from jax.experimental import pallas as pl
from jax.experimental.pallas import tpu as pltpu
from jax.experimental.pallas import tpu_sc as plsc

N_DAYS = 366
D_MODEL = 256
BATCH = 16384

NC = 2
NS = 16
NW = NC * NS
B_PER_W = BATCH // NW
CHUNK = 128
N_CHUNK = B_PER_W // CHUNK
NBUF = 3
NREP = 8
N_DAYS_PAD = 368

_mesh = plsc.VectorSubcoreMesh(core_axis_name="c", subcore_axis_name="s")


@functools.partial(
    pl.kernel,
    mesh=_mesh,
    out_type=jax.ShapeDtypeStruct((BATCH, D_MODEL), jnp.float32),
    scratch_types=[
        pltpu.VMEM((N_CHUNK, CHUNK), jnp.int32),
        *[pltpu.VMEM((CHUNK, D_MODEL), jnp.float32) for _ in range(NBUF)],
        pltpu.SemaphoreType.DMA,
        *[pltpu.SemaphoreType.DMA for _ in range(2 * NBUF)],
    ],
)
def _gather_kernel(x_hbm, repl_hbm, out_hbm, idx_v, *scratch):
    bufs = scratch[:NBUF]
    isem = scratch[NBUF]
    gsems = scratch[NBUF + 1:NBUF + 1 + NBUF]
    ssems = scratch[NBUF + 1 + NBUF:]
    sid = lax.axis_index("s")
    wid = sid * NC + lax.axis_index("c")
    base = wid * B_PER_W

    icp = [
        pltpu.async_copy(x_hbm.at[pl.ds(base + j * CHUNK, CHUNK)], idx_v.at[j], isem)
        for j in range(N_CHUNK)
    ]
    for cp in icp:
        cp.wait()

    roff = (wid % NREP) * N_DAYS_PAD
    for j in range(N_CHUNK):
        for k in range(CHUNK // 16):
            v = idx_v[j, pl.ds(k * 16, 16)]
            idx_v[j, pl.ds(k * 16, 16)] = jnp.where(v == 0, N_DAYS - 1, v - 1) + roff

    def gather(j):
        return pltpu.async_copy(repl_hbm.at[idx_v.at[j]], bufs[j % NBUF], gsems[j % NBUF])

    def scatter(j):
        return pltpu.async_copy(
            bufs[j % NBUF], out_hbm.at[pl.ds(base + j * CHUNK, CHUNK)], ssems[j % NBUF]
        )

    gcp = [None] * N_CHUNK
    scp = [None] * N_CHUNK
    for j in range(min(NBUF, N_CHUNK)):
        gcp[j] = gather(j)
    for j in range(N_CHUNK):
        gcp[j].wait()
        scp[j] = scatter(j)
        if j + NBUF < N_CHUNK:
            scp[j].wait()
            gcp[j + NBUF] = gather(j + NBUF)
    for j in range(max(0, N_CHUNK - NBUF), N_CHUNK):
        scp[j].wait()


def kernel(x, pe):
    pe_pad = jnp.pad(pe, ((0, N_DAYS_PAD - N_DAYS), (0, 0)))
    repl = jnp.tile(pe_pad, (NREP, 1))
    return _gather_kernel(x.astype(jnp.int32), repl)
